# baseline (speedup 1.0000x reference)
import functools

import jax
import jax.numpy as jnp
from jax import lax
from jax.experimental import pallas as pl
from jax.experimental.pallas import tpu as pltpu
from jax.experimental.pallas import tpu_sc as plsc

_N = 10000
_E = 160000
_HP = 320
_HC = _HP // 2
_G = 64
_CH = 128
_NCHUNK = _E // _CH
_SCH = 64
_NCHUNK_S = _E // _SCH
_NW = 32
_ZSTR = 624
_ZCH = 640
_FCH = 632


def _sc_gather(tables, idx2d, widths):
    nt = len(tables)
    per_w = _NCHUNK // _NW
    rem = _NCHUNK - per_w * _NW
    mesh = plsc.VectorSubcoreMesh(core_axis_name="c", subcore_axis_name="s")

    dts = [t.dtype for t in tables]
    scratch = [pltpu.VMEM((per_w + 1, _CH), jnp.int32)]
    for t in range(nt):
        for b in range(2):
            scratch.append(pltpu.VMEM((_CH, widths[t]), dts[t]))
    scratch.append(pltpu.SemaphoreType.DMA)
    scratch += [pltpu.SemaphoreType.DMA, pltpu.SemaphoreType.DMA]

    @functools.partial(
        pl.kernel,
        out_type=[jax.ShapeDtypeStruct((_E, w), dt)
                  for w, dt in zip(widths, dts)],
        mesh=mesh,
        scratch_types=scratch,
        compiler_params=pltpu.CompilerParams(use_tc_tiling_on_sc=False),
    )
    def k(*refs):
        tabs = refs[:nt]
        idx_hbm = refs[nt]
        outs = refs[nt + 1:nt + 1 + nt]
        idx_v = refs[nt + 1 + nt]
        bufs = [[refs[nt + 2 + nt + 2 * t + b] for b in range(2)]
                for t in range(nt)]
        gsem = refs[-3]
        wsem = [refs[-2], refs[-1]]

        c = lax.axis_index("c")
        s = lax.axis_index("s")
        w = s * 2 + c
        start = per_w * w + jnp.minimum(w, rem)
        count = per_w + (w < rem).astype(jnp.int32)

        pltpu.sync_copy(idx_hbm.at[pl.ds(start, per_w)],
                        idx_v.at[pl.ds(0, per_w)])

        @pl.when(w < rem)
        def _():
            pltpu.sync_copy(idx_hbm.at[pl.ds(start + per_w, 1)],
                            idx_v.at[pl.ds(per_w, 1)])

        def chunk(k_, b):
            @pl.when(k_ < count)
            def _():
                @pl.when(k_ >= 2)
                def _():
                    for t in range(nt):
                        pltpu.make_async_copy(
                            bufs[t][b], outs[t].at[pl.ds(0, _CH)],
                            wsem[b]).wait()

                row = idx_v.at[k_]
                descs = [
                    pltpu.async_copy(tabs[t].at[row], bufs[t][b], gsem)
                    for t in range(nt)
                ]
                for dsc in descs:
                    dsc.wait()
                base = (start + k_) * _CH
                for t in range(nt):
                    pltpu.async_copy(bufs[t][b],
                                     outs[t].at[pl.ds(base, _CH)], wsem[b])

        def outer(k2, carry):
            chunk(k2 * 2, 0)
            chunk(k2 * 2 + 1, 1)
            return carry

        lax.fori_loop(0, (per_w + 2) // 2, outer, 0)

        for b in range(2):
            for t in range(nt):
                pltpu.make_async_copy(bufs[t][b], outs[t].at[pl.ds(0, _CH)],
                                      wsem[b]).wait()

    return k(*tables, idx2d)


def _sc_scatter_add(msg2, idx2d, zrows):
    per_t = _NCHUNK_S // 16
    rem = _NCHUNK_S - per_t * 16
    mesh = plsc.VectorSubcoreMesh(core_axis_name="c", subcore_axis_name="s")

    @functools.partial(
        pl.kernel,
        out_type=[jax.ShapeDtypeStruct((_N, _HC), jnp.float32),
                  jax.ShapeDtypeStruct((_N, _HC), jnp.float32)],
        mesh=mesh,
        scratch_types=[
            pltpu.VMEM((per_t + 1, _SCH), jnp.int32),
            pltpu.VMEM((_SCH, _HC), jnp.float32),
            pltpu.VMEM((_SCH, _HC), jnp.float32),
            pltpu.VMEM_SHARED((_N, _HC), jnp.float32),
            pltpu.SemaphoreType.DMA,
            pltpu.SemaphoreType.DMA,
            pltpu.SemaphoreType.DMA,
            pltpu.SemaphoreType.DMA,
        ],
        compiler_params=pltpu.CompilerParams(use_tc_tiling_on_sc=False),
    )
    def k(msg_hbm, idx_hbm, z_hbm, outL, outR, idx_v, mv0, mv1, acc,
          l0, l1, a0, a1):
        c = lax.axis_index("c")
        s = lax.axis_index("s")
        mv = [mv0, mv1]
        lsem = [l0, l1]
        asem = [a0, a1]
        start = per_t * s + jnp.minimum(s, rem)
        count = per_t + (s < rem).astype(jnp.int32)

        pltpu.sync_copy(z_hbm, acc.at[pl.ds(s * _ZSTR, _ZCH)])

        pltpu.sync_copy(idx_hbm.at[pl.ds(start, per_t)],
                        idx_v.at[pl.ds(0, per_t)])

        @pl.when(s < rem)
        def _():
            pltpu.sync_copy(idx_hbm.at[pl.ds(start + per_t, 1)],
                            idx_v.at[pl.ds(per_t, 1)])

        plsc.subcore_barrier()

        def chunk(k_, b):
            @pl.when(k_ < count)
            def _():
                @pl.when(k_ >= 2)
                def _():
                    pltpu.make_async_copy(mv[b], acc.at[pl.ds(0, _SCH)],
                                          asem[b]).wait()
                pltpu.async_copy(
                    msg_hbm.at[c, pl.ds((start + k_) * _SCH, _SCH)],
                    mv[b], lsem[b])
                pltpu.make_async_copy(
                    msg_hbm.at[c, pl.ds(0, _SCH)], mv[b], lsem[b]).wait()
                pltpu.async_copy(mv[b], acc.at[idx_v.at[k_]], asem[b],
                                 add=True)

        def outer(k2, carry):
            chunk(k2 * 2, 0)
            chunk(k2 * 2 + 1, 1)
            return carry

        lax.fori_loop(0, (per_t + 2) // 2, outer, 0)

        for b in range(2):
            pltpu.make_async_copy(mv[b], acc.at[pl.ds(0, _SCH)],
                                  asem[b]).wait()

        plsc.subcore_barrier()

        @pl.when(c == 0)
        def _():
            @pl.when(s < 15)
            def _():
                pltpu.sync_copy(acc.at[pl.ds(s * _FCH, _FCH)],
                                outL.at[pl.ds(s * _FCH, _FCH)])

            @pl.when(s == 15)
            def _():
                pltpu.sync_copy(acc.at[pl.ds(15 * _FCH, _N - 15 * _FCH)],
                                outL.at[pl.ds(15 * _FCH, _N - 15 * _FCH)])

        @pl.when(c == 1)
        def _():
            @pl.when(s < 15)
            def _():
                pltpu.sync_copy(acc.at[pl.ds(s * _FCH, _FCH)],
                                outR.at[pl.ds(s * _FCH, _FCH)])

            @pl.when(s == 15)
            def _():
                pltpu.sync_copy(acc.at[pl.ds(15 * _FCH, _N - 15 * _FCH)],
                                outR.at[pl.ds(15 * _FCH, _N - 15 * _FCH)])

    return k(msg2, idx2d, zrows)


_BE = 640
_GRID_E = _E // _BE


def _silu(x):
    return x * jax.nn.sigmoid(x)


def _split_store(m2_ref, m):
    m2_ref[0] = m[:, :_HC]
    m2_ref[1] = m[:, _HC:]


def _tc_edge_init(xg, ea, WeX, WeE, be2, W0, b0):
    def body(xg_ref, ea_ref, wx_ref, we_ref, be_ref, w0_ref, b0_ref,
             e_ref, m_ref):
        xgf = xg_ref[...].astype(jnp.float32)
        h = jnp.dot(xgf, wx_ref[...], preferred_element_type=jnp.float32)
        h = h + jnp.dot(ea_ref[...], we_ref[...], preferred_element_type=jnp.float32)
        h = h + be_ref[...]
        e = _silu(h)
        e_ref[...] = e.astype(jnp.bfloat16)
        m = jnp.dot(e, w0_ref[...], preferred_element_type=jnp.float32) + b0_ref[...]
        _split_store(m_ref, jnp.maximum(m, 0.0))

    full = lambda shape: pl.BlockSpec(shape, lambda i: (0, 0))
    return pl.pallas_call(
        body,
        grid=(_GRID_E,),
        in_specs=[
            pl.BlockSpec((_BE, 128), lambda i: (i, 0)),
            pl.BlockSpec((_BE, 16), lambda i: (i, 0)),
            full((128, _HP)),
            full((16, _HP)),
            full((1, _HP)),
            full((_HP, _HP)),
            full((1, _HP)),
        ],
        out_specs=[
            pl.BlockSpec((_BE, _HP), lambda i: (i, 0)),
            pl.BlockSpec((2, _BE, _HC), lambda i: (0, i, 0)),
        ],
        out_shape=[
            jax.ShapeDtypeStruct((_E, _HP), jnp.bfloat16),
            jax.ShapeDtypeStruct((2, _E, _HC), jnp.float32),
        ],
        compiler_params=pltpu.CompilerParams(
            dimension_semantics=("parallel",)),
    )(xg, ea, WeX, WeE, be2, W0, b0)


def _tc_layer(agL, agR, ep, mW, mb2, nW, nb2, last):
    def body(agl_ref, agr_ref, ep_ref, mw_ref, mb_ref, nw_ref, nb_ref, *outs):
        ag = jnp.concatenate([agl_ref[...], agr_ref[...]], axis=1)
        d = ag.astype(jnp.float32) - ep_ref[...].astype(jnp.float32)
        h = jnp.dot(d, mw_ref[...], preferred_element_type=jnp.float32) + mb_ref[...]
        h = jnp.maximum(h, 0.0)
        if last:
            e = 2.0 * h
            m_ref, = outs
        else:
            e = _silu(h) + h
            e_ref, m_ref = outs
            e_ref[...] = e.astype(jnp.bfloat16)
        m = jnp.dot(e, nw_ref[...], preferred_element_type=jnp.float32) + nb_ref[...]
        _split_store(m_ref, jnp.maximum(m, 0.0))

    full = lambda shape: pl.BlockSpec(shape, lambda i: (0, 0))
    out_specs = [pl.BlockSpec((2, _BE, _HC), lambda i: (0, i, 0))]
    out_shape = [jax.ShapeDtypeStruct((2, _E, _HC), jnp.float32)]
    if not last:
        out_specs = [pl.BlockSpec((_BE, _HP), lambda i: (i, 0))] + out_specs
        out_shape = [jax.ShapeDtypeStruct((_E, _HP), jnp.bfloat16)] + out_shape
    return pl.pallas_call(
        body,
        grid=(_GRID_E,),
        in_specs=[
            pl.BlockSpec((_BE, _HC), lambda i: (i, 0)),
            pl.BlockSpec((_BE, _HC), lambda i: (i, 0)),
            pl.BlockSpec((_BE, _HP), lambda i: (i, 0)),
            full((_HP, _HP)),
            full((1, _HP)),
            full((_HP, _HP)),
            full((1, _HP)),
        ],
        out_specs=out_specs,
        out_shape=out_shape,
        compiler_params=pltpu.CompilerParams(
            dimension_semantics=("parallel",)),
    )(agL, agR, ep, mW, mb2, nW, nb2)


_BN = 400
_GRID_N = _N // _BN


def _tc_pool_ffn(nhL, nhR, batch3, W1p, b12, W2, b22, W3, b32):
    def body(nhl_ref, nhr_ref, b_ref, w1_ref, b1_ref, w2_ref, b2_ref,
             w3_ref, b3_ref, out_ref, acc):
        i = pl.program_id(0)
        nh = jnp.concatenate([nhl_ref[...], nhr_ref[...]], axis=1)
        seg = jnp.broadcast_to(b_ref[0], (_G, _BN))
        gids = lax.broadcasted_iota(jnp.int32, (_G, _BN), 0)
        onehot = (seg == gids).astype(jnp.float32)
        part = jnp.dot(onehot, nh, preferred_element_type=jnp.float32)

        @pl.when(i == 0)
        def _():
            acc[...] = jnp.zeros_like(acc)

        acc[...] += part

        @pl.when(i == _GRID_N - 1)
        def _():
            h = jnp.dot(acc[...], w1_ref[...], preferred_element_type=jnp.float32)
            h = _silu(h + b1_ref[...])
            h = jnp.dot(h, w2_ref[...], preferred_element_type=jnp.float32)
            h = _silu(h + b2_ref[...])
            h = jnp.dot(h, w3_ref[...], preferred_element_type=jnp.float32)
            out_ref[...] = h + b3_ref[...]

    full = lambda shape: pl.BlockSpec(shape, lambda i: tuple(0 for _ in shape))
    return pl.pallas_call(
        body,
        grid=(_GRID_N,),
        in_specs=[
            pl.BlockSpec((_BN, _HC), lambda i: (i, 0)),
            pl.BlockSpec((_BN, _HC), lambda i: (i, 0)),
            pl.BlockSpec((1, 1, _BN), lambda i: (i, 0, 0)),
            full((_HP, 300)),
            full((1, 300)),
            full((300, 300)),
            full((1, 300)),
            full((300, 1)),
            full((1, 1)),
        ],
        out_specs=pl.BlockSpec((_G, 1), lambda i: (0, 0)),
        out_shape=jax.ShapeDtypeStruct((_G, 1), jnp.float32),
        scratch_shapes=[pltpu.VMEM((_G, _HP), jnp.float32)],
        compiler_params=pltpu.CompilerParams(
            dimension_semantics=("arbitrary",)),
    )(nhL, nhR, batch3, W1p, b12, W2, b22, W3, b32)


def kernel(x, edge_index, edge_attr, batch, We, be, linW, linb, mlpW, mlpb,
           n2W, n2b, W1, b1, W2, b2, W3, b3):
    f32 = jnp.float32
    row = edge_index[0].astype(jnp.int32)
    col = edge_index[1].astype(jnp.int32)
    row_sw = row.reshape(_E // 2, 2)[:, ::-1].reshape(_NCHUNK, _CH)
    col_sw = col.reshape(_E // 2, 2)[:, ::-1].reshape(_NCHUNK_S, _SCH)
    row = row.reshape(_NCHUNK, _CH)
    col = col.reshape(_NCHUNK_S, _SCH)

    pad_h = lambda w: jnp.pad(w, ((0, _HP - w.shape[0]), (0, _HP - w.shape[1])))
    pad_b = lambda v: jnp.pad(v, (0, _HP - v.shape[0])).reshape(1, _HP)

    WeX = jnp.pad(We[:128], ((0, 0), (0, _HP - 300)))
    WeE = jnp.pad(We[128:], ((0, 0), (0, _HP - 300)))
    be2 = pad_b(be)
    linWp = [pad_h(linW[l]) for l in range(3)]
    linbp = [pad_b(linb[l]) for l in range(3)]
    mlpWp = [pad_h(mlpW[l]) for l in range(3)]
    mlpbp = [pad_b(mlpb[l]) for l in range(3)]
    n2Wp = pad_h(n2W)
    n2bp = pad_b(n2b)
    W1p = jnp.pad(W1, ((0, _HP - 300), (0, 0)))
    b12 = b1.reshape(1, 300)
    b22 = b2.reshape(1, 300)
    b32 = b3.reshape(1, 1)
    batch3 = batch.astype(jnp.int32).reshape(_GRID_N, 1, _BN)
    zrows = jnp.zeros((_ZCH, _HC), f32)

    bf16 = jnp.bfloat16

    xg, = _sc_gather([x.astype(bf16)], row, [128])
    e0, msg0 = _tc_edge_init(xg, edge_attr, WeX, WeE, be2, linWp[0], linbp[0])

    a0L, a0R = _sc_scatter_add(msg0, col, zrows)
    ag0L, ag0R = _sc_gather([a0L.astype(bf16), a0R.astype(bf16)],
                            row_sw, [_HC, _HC])
    e1s, msg1s = _tc_layer(ag0L, ag0R, e0, mlpWp[0], mlpbp[0],
                           linWp[1], linbp[1], False)

    a1L, a1R = _sc_scatter_add(msg1s, col_sw, zrows)
    ag1L, ag1R = _sc_gather([a1L.astype(bf16), a1R.astype(bf16)],
                            row, [_HC, _HC])
    e2, msg2 = _tc_layer(ag1L, ag1R, e1s, mlpWp[1], mlpbp[1],
                         linWp[2], linbp[2], False)

    a2L, a2R = _sc_scatter_add(msg2, col, zrows)
    ag2L, ag2R = _sc_gather([a2L.astype(bf16), a2R.astype(bf16)],
                            row_sw, [_HC, _HC])
    msgf, = _tc_layer(ag2L, ag2R, e2, mlpWp[2], mlpbp[2], n2Wp, n2bp, True)

    nhL, nhR = _sc_scatter_add(msgf, col_sw, zrows)
    return _tc_pool_ffn(nhL, nhR, batch3, W1p, b12, W2, b22, W3, b32)


# speedup vs baseline: 1.1151x; 1.1151x over previous
import functools

import jax
import jax.numpy as jnp
from jax import lax
from jax.experimental import pallas as pl
from jax.experimental.pallas import tpu as pltpu
from jax.experimental.pallas import tpu_sc as plsc

_N = 10000
_E = 160000
_HP = 304
_HC = _HP // 2
_G = 64
_CH = 128
_NCHUNK = _E // _CH
_SCH = 64
_NCHUNK_S = _E // _SCH
_NW = 32
_ZSTR = 624
_ZCH = 640
_FCH = 632


def _swap_pairs(idx_v, n):
    par = (jax.lax.iota(jnp.int32, 16) & 1) == 0

    def s16(i, carry):
        o = 8 + i * 16
        a = idx_v[pl.ds(o + 1, 16)]
        b = idx_v[pl.ds(o - 1, 16)]
        idx_v[pl.ds(o, 16)] = jnp.where(par, a, b)
        return carry

    lax.fori_loop(0, n // 16, s16, 0)


def _sc_gather(tables, ei3, d0, swap, widths):
    nt = len(tables)
    per_w = _NCHUNK // _NW
    rem = _NCHUNK - per_w * _NW
    mesh = plsc.VectorSubcoreMesh(core_axis_name="c", subcore_axis_name="s")

    scratch = [pltpu.VMEM(((per_w + 1) * _CH + 16,), jnp.int32)]
    for t in range(nt):
        for b in range(2):
            scratch.append(pltpu.VMEM((_CH, widths[t]), jnp.float32))
    scratch.append(pltpu.SemaphoreType.DMA)
    scratch += [pltpu.SemaphoreType.DMA, pltpu.SemaphoreType.DMA]

    @functools.partial(
        pl.kernel,
        out_type=[jax.ShapeDtypeStruct((_E, w), jnp.float32) for w in widths],
        mesh=mesh,
        scratch_types=scratch,
        compiler_params=pltpu.CompilerParams(use_tc_tiling_on_sc=False),
    )
    def k(*refs):
        tabs = refs[:nt]
        idx_hbm = refs[nt]
        outs = refs[nt + 1:nt + 1 + nt]
        idx_v = refs[nt + 1 + nt]
        bufs = [[refs[nt + 2 + nt + 2 * t + b] for b in range(2)]
                for t in range(nt)]
        gsem = refs[-3]
        wsem = [refs[-2], refs[-1]]

        c = lax.axis_index("c")
        s = lax.axis_index("s")
        w = s * 2 + c
        start = per_w * w + jnp.minimum(w, rem)
        count = per_w + (w < rem).astype(jnp.int32)

        pltpu.sync_copy(idx_hbm.at[d0, pl.ds(start * _CH, per_w * _CH)],
                        idx_v.at[pl.ds(8, per_w * _CH)])

        @pl.when(w < rem)
        def _():
            pltpu.sync_copy(
                idx_hbm.at[d0, pl.ds((start + per_w) * _CH, _CH)],
                idx_v.at[pl.ds(8 + per_w * _CH, _CH)])

        if swap:
            _swap_pairs(idx_v, (per_w + 1) * _CH)

        def chunk(k_, b):
            @pl.when(k_ < count)
            def _():
                @pl.when(k_ >= 2)
                def _():
                    for t in range(nt):
                        pltpu.make_async_copy(
                            bufs[t][b], outs[t].at[pl.ds(0, _CH)],
                            wsem[b]).wait()

                row = idx_v.at[pl.ds(8 + k_ * _CH, _CH)]
                descs = [
                    pltpu.async_copy(tabs[t].at[row], bufs[t][b], gsem)
                    for t in range(nt)
                ]
                for dsc in descs:
                    dsc.wait()
                base = (start + k_) * _CH
                for t in range(nt):
                    pltpu.async_copy(bufs[t][b],
                                     outs[t].at[pl.ds(base, _CH)], wsem[b])

        def outer(k2, carry):
            chunk(k2 * 2, 0)
            chunk(k2 * 2 + 1, 1)
            return carry

        lax.fori_loop(0, (per_w + 2) // 2, outer, 0)

        for b in range(2):
            for t in range(nt):
                pltpu.make_async_copy(bufs[t][b], outs[t].at[pl.ds(0, _CH)],
                                      wsem[b]).wait()

    return k(*tables, ei3)


def _sc_scatter_add(msg2, ei3s, swap, zrows):
    per_t = _NCHUNK_S // 16
    rem = _NCHUNK_S - per_t * 16
    mesh = plsc.VectorSubcoreMesh(core_axis_name="c", subcore_axis_name="s")

    @functools.partial(
        pl.kernel,
        out_type=[jax.ShapeDtypeStruct((_N, _HC), jnp.float32),
                  jax.ShapeDtypeStruct((_N, _HC), jnp.float32)],
        mesh=mesh,
        scratch_types=[
            pltpu.VMEM(((per_t + 1) * _SCH + 16,), jnp.int32),
            pltpu.VMEM((_SCH, _HC), jnp.float32),
            pltpu.VMEM((_SCH, _HC), jnp.float32),
            pltpu.VMEM_SHARED((_N, _HC), jnp.float32),
            pltpu.SemaphoreType.DMA,
            pltpu.SemaphoreType.DMA,
            pltpu.SemaphoreType.DMA,
            pltpu.SemaphoreType.DMA,
        ],
        compiler_params=pltpu.CompilerParams(use_tc_tiling_on_sc=False),
    )
    def k(msg_hbm, idx_hbm, z_hbm, outL, outR, idx_v, mv0, mv1, acc,
          l0, l1, a0, a1):
        c = lax.axis_index("c")
        s = lax.axis_index("s")
        mv = [mv0, mv1]
        lsem = [l0, l1]
        asem = [a0, a1]
        start = per_t * s + jnp.minimum(s, rem)
        count = per_t + (s < rem).astype(jnp.int32)

        pltpu.sync_copy(z_hbm, acc.at[pl.ds(s * _ZSTR, _ZCH)])

        pltpu.sync_copy(idx_hbm.at[1, pl.ds(start * _SCH, per_t * _SCH)],
                        idx_v.at[pl.ds(8, per_t * _SCH)])

        @pl.when(s < rem)
        def _():
            pltpu.sync_copy(
                idx_hbm.at[1, pl.ds((start + per_t) * _SCH, _SCH)],
                idx_v.at[pl.ds(8 + per_t * _SCH, _SCH)])

        if swap:
            _swap_pairs(idx_v, (per_t + 1) * _SCH)

        plsc.subcore_barrier()

        def chunk(k_, b):
            @pl.when(k_ < count)
            def _():
                @pl.when(k_ >= 2)
                def _():
                    pltpu.make_async_copy(mv[b], acc.at[pl.ds(0, _SCH)],
                                          asem[b]).wait()
                pltpu.async_copy(
                    msg_hbm.at[c, pl.ds((start + k_) * _SCH, _SCH)],
                    mv[b], lsem[b])
                pltpu.make_async_copy(
                    msg_hbm.at[c, pl.ds(0, _SCH)], mv[b], lsem[b]).wait()
                pltpu.async_copy(
                    mv[b], acc.at[idx_v.at[pl.ds(8 + k_ * _SCH, _SCH)]],
                    asem[b], add=True)

        def outer(k2, carry):
            chunk(k2 * 2, 0)
            chunk(k2 * 2 + 1, 1)
            return carry

        lax.fori_loop(0, (per_t + 2) // 2, outer, 0)

        for b in range(2):
            pltpu.make_async_copy(mv[b], acc.at[pl.ds(0, _SCH)],
                                  asem[b]).wait()

        plsc.subcore_barrier()

        @pl.when(c == 0)
        def _():
            @pl.when(s < 15)
            def _():
                pltpu.sync_copy(acc.at[pl.ds(s * _FCH, _FCH)],
                                outL.at[pl.ds(s * _FCH, _FCH)])

            @pl.when(s == 15)
            def _():
                pltpu.sync_copy(acc.at[pl.ds(15 * _FCH, _N - 15 * _FCH)],
                                outL.at[pl.ds(15 * _FCH, _N - 15 * _FCH)])

        @pl.when(c == 1)
        def _():
            @pl.when(s < 15)
            def _():
                pltpu.sync_copy(acc.at[pl.ds(s * _FCH, _FCH)],
                                outR.at[pl.ds(s * _FCH, _FCH)])

            @pl.when(s == 15)
            def _():
                pltpu.sync_copy(acc.at[pl.ds(15 * _FCH, _N - 15 * _FCH)],
                                outR.at[pl.ds(15 * _FCH, _N - 15 * _FCH)])

    return k(msg2, ei3s, zrows)


_BE = 640
_GRID_E = _E // _BE


def _silu(x):
    return x * jax.nn.sigmoid(x)


def _split_store(m2_ref, m):
    m2_ref[0] = m[:, :_HC]
    m2_ref[1] = m[:, _HC:]


def _tc_edge_init(xg, ea, WeX, WeE, be2, W0, b0):
    def body(xg_ref, ea_ref, wx_ref, we_ref, be_ref, w0_ref, b0_ref,
             e_ref, m_ref):
        h = jnp.dot(xg_ref[...], wx_ref[...], preferred_element_type=jnp.float32)
        h = h + jnp.dot(ea_ref[...], we_ref[...], preferred_element_type=jnp.float32)
        h = h + be_ref[...]
        e = _silu(h)
        e_ref[...] = e
        m = jnp.dot(e, w0_ref[...], preferred_element_type=jnp.float32) + b0_ref[...]
        _split_store(m_ref, jnp.maximum(m, 0.0))

    full = lambda shape: pl.BlockSpec(shape, lambda i: (0, 0))
    return pl.pallas_call(
        body,
        grid=(_GRID_E,),
        in_specs=[
            pl.BlockSpec((_BE, 128), lambda i: (i, 0)),
            pl.BlockSpec((_BE, 16), lambda i: (i, 0)),
            full((128, _HP)),
            full((16, _HP)),
            full((1, _HP)),
            full((_HP, _HP)),
            full((1, _HP)),
        ],
        out_specs=[
            pl.BlockSpec((_BE, _HP), lambda i: (i, 0)),
            pl.BlockSpec((2, _BE, _HC), lambda i: (0, i, 0)),
        ],
        out_shape=[
            jax.ShapeDtypeStruct((_E, _HP), jnp.float32),
            jax.ShapeDtypeStruct((2, _E, _HC), jnp.float32),
        ],
        compiler_params=pltpu.CompilerParams(
            dimension_semantics=("parallel",)),
    )(xg, ea, WeX, WeE, be2, W0, b0)


def _tc_layer(agL, agR, ep, mW, mb2, nW, nb2, last):
    def body(agl_ref, agr_ref, ep_ref, mw_ref, mb_ref, nw_ref, nb_ref, *outs):
        ag = jnp.concatenate([agl_ref[...], agr_ref[...]], axis=1)
        d = ag - ep_ref[...]
        h = jnp.dot(d, mw_ref[...], preferred_element_type=jnp.float32) + mb_ref[...]
        h = jnp.maximum(h, 0.0)
        if last:
            e = 2.0 * h
            m_ref, = outs
        else:
            e = _silu(h) + h
            e_ref, m_ref = outs
            e_ref[...] = e
        m = jnp.dot(e, nw_ref[...], preferred_element_type=jnp.float32) + nb_ref[...]
        _split_store(m_ref, jnp.maximum(m, 0.0))

    full = lambda shape: pl.BlockSpec(shape, lambda i: (0, 0))
    out_specs = [pl.BlockSpec((2, _BE, _HC), lambda i: (0, i, 0))]
    out_shape = [jax.ShapeDtypeStruct((2, _E, _HC), jnp.float32)]
    if not last:
        out_specs = [pl.BlockSpec((_BE, _HP), lambda i: (i, 0))] + out_specs
        out_shape = [jax.ShapeDtypeStruct((_E, _HP), jnp.float32)] + out_shape
    return pl.pallas_call(
        body,
        grid=(_GRID_E,),
        in_specs=[
            pl.BlockSpec((_BE, _HC), lambda i: (i, 0)),
            pl.BlockSpec((_BE, _HC), lambda i: (i, 0)),
            pl.BlockSpec((_BE, _HP), lambda i: (i, 0)),
            full((_HP, _HP)),
            full((1, _HP)),
            full((_HP, _HP)),
            full((1, _HP)),
        ],
        out_specs=out_specs,
        out_shape=out_shape,
        compiler_params=pltpu.CompilerParams(
            dimension_semantics=("parallel",)),
    )(agL, agR, ep, mW, mb2, nW, nb2)


_BN = 400
_GRID_N = _N // _BN


def _tc_pool_ffn(nhL, nhR, batch3, W1p, b12, W2, b22, W3, b32):
    def body(nhl_ref, nhr_ref, b_ref, w1_ref, b1_ref, w2_ref, b2_ref,
             w3_ref, b3_ref, out_ref, acc):
        i = pl.program_id(0)
        nh = jnp.concatenate([nhl_ref[...], nhr_ref[...]], axis=1)
        seg = jnp.broadcast_to(b_ref[0], (_G, _BN))
        gids = lax.broadcasted_iota(jnp.int32, (_G, _BN), 0)
        onehot = (seg == gids).astype(jnp.float32)
        part = jnp.dot(onehot, nh, preferred_element_type=jnp.float32)

        @pl.when(i == 0)
        def _():
            acc[...] = jnp.zeros_like(acc)

        acc[...] += part

        @pl.when(i == _GRID_N - 1)
        def _():
            h = jnp.dot(acc[...], w1_ref[...], preferred_element_type=jnp.float32)
            h = _silu(h + b1_ref[...])
            h = jnp.dot(h, w2_ref[...], preferred_element_type=jnp.float32)
            h = _silu(h + b2_ref[...])
            h = jnp.dot(h, w3_ref[...], preferred_element_type=jnp.float32)
            out_ref[...] = h + b3_ref[...]

    full = lambda shape: pl.BlockSpec(shape, lambda i: tuple(0 for _ in shape))
    return pl.pallas_call(
        body,
        grid=(_GRID_N,),
        in_specs=[
            pl.BlockSpec((_BN, _HC), lambda i: (i, 0)),
            pl.BlockSpec((_BN, _HC), lambda i: (i, 0)),
            pl.BlockSpec((1, 1, _BN), lambda i: (i, 0, 0)),
            full((_HP, 300)),
            full((1, 300)),
            full((300, 300)),
            full((1, 300)),
            full((300, 1)),
            full((1, 1)),
        ],
        out_specs=pl.BlockSpec((_G, 1), lambda i: (0, 0)),
        out_shape=jax.ShapeDtypeStruct((_G, 1), jnp.float32),
        scratch_shapes=[pltpu.VMEM((_G, _HP), jnp.float32)],
        compiler_params=pltpu.CompilerParams(
            dimension_semantics=("arbitrary",)),
    )(nhL, nhR, batch3, W1p, b12, W2, b22, W3, b32)


def kernel(x, edge_index, edge_attr, batch, We, be, linW, linb, mlpW, mlpb,
           n2W, n2b, W1, b1, W2, b2, W3, b3):
    f32 = jnp.float32
    ei = edge_index.astype(jnp.int32)
    ei3 = ei
    ei3s = ei

    pad_h = lambda w: jnp.pad(w, ((0, _HP - w.shape[0]), (0, _HP - w.shape[1])))
    pad_b = lambda v: jnp.pad(v, (0, _HP - v.shape[0])).reshape(1, _HP)

    WeX = jnp.pad(We[:128], ((0, 0), (0, _HP - 300)))
    WeE = jnp.pad(We[128:], ((0, 0), (0, _HP - 300)))
    be2 = pad_b(be)
    linWp = [pad_h(linW[l]) for l in range(3)]
    linbp = [pad_b(linb[l]) for l in range(3)]
    mlpWp = [pad_h(mlpW[l]) for l in range(3)]
    mlpbp = [pad_b(mlpb[l]) for l in range(3)]
    n2Wp = pad_h(n2W)
    n2bp = pad_b(n2b)
    W1p = jnp.pad(W1, ((0, _HP - 300), (0, 0)))
    b12 = b1.reshape(1, 300)
    b22 = b2.reshape(1, 300)
    b32 = b3.reshape(1, 1)
    batch3 = batch.astype(jnp.int32).reshape(_GRID_N, 1, _BN)
    zrows = jnp.zeros((_ZCH, _HC), f32)

    xg, = _sc_gather([x], ei3, 0, False, [128])
    e0, msg0 = _tc_edge_init(xg, edge_attr, WeX, WeE, be2, linWp[0], linbp[0])

    a0L, a0R = _sc_scatter_add(msg0, ei3s, False, zrows)
    ag0L, ag0R = _sc_gather([a0L, a0R], ei3, 0, True, [_HC, _HC])
    e1s, msg1s = _tc_layer(ag0L, ag0R, e0, mlpWp[0], mlpbp[0],
                           linWp[1], linbp[1], False)

    a1L, a1R = _sc_scatter_add(msg1s, ei3s, True, zrows)
    ag1L, ag1R = _sc_gather([a1L, a1R], ei3, 0, False, [_HC, _HC])
    e2, msg2 = _tc_layer(ag1L, ag1R, e1s, mlpWp[1], mlpbp[1],
                         linWp[2], linbp[2], False)

    a2L, a2R = _sc_scatter_add(msg2, ei3s, False, zrows)
    ag2L, ag2R = _sc_gather([a2L, a2R], ei3, 0, True, [_HC, _HC])
    msgf, = _tc_layer(ag2L, ag2R, e2, mlpWp[2], mlpbp[2], n2Wp, n2bp, True)

    nhL, nhR = _sc_scatter_add(msgf, ei3s, True, zrows)
    return _tc_pool_ffn(nhL, nhR, batch3, W1p, b12, W2, b22, W3, b32)


# speedup vs baseline: 1.8324x; 1.6434x over previous
import functools

import jax
import jax.numpy as jnp
from jax import lax
from jax.experimental import pallas as pl
from jax.experimental.pallas import tpu as pltpu
from jax.experimental.pallas import tpu_sc as plsc

_N = 10000
_E = 160000
_PW = 128
_NP = 3
_HP = _PW * _NP
_G = 64
_CH = 128
_NCHUNK = _E // _CH
_SCH = 64
_NCHUNK_S = _E // _SCH
_NW = 32
_ZSTR = 624
_ZCH = 640
_FCH = 632
_OWNER = (0, 1, 0)


def _swap_pairs(idx_v, n):
    par = (jax.lax.iota(jnp.int32, 16) & 1) == 0

    def s16(i, carry):
        o = 8 + i * 16
        a = idx_v[pl.ds(o + 1, 16)]
        b = idx_v[pl.ds(o - 1, 16)]
        idx_v[pl.ds(o, 16)] = jnp.where(par, a, b)
        return carry

    lax.fori_loop(0, n // 16, s16, 0)


def _sc_gather(tables, ei, d0, swap, widths):
    nt = len(tables)
    per_w = _NCHUNK // _NW
    rem = _NCHUNK - per_w * _NW
    mesh = plsc.VectorSubcoreMesh(core_axis_name="c", subcore_axis_name="s")

    scratch = [pltpu.VMEM(((per_w + 1) * _CH + 16,), jnp.int32)]
    for t in range(nt):
        for b in range(2):
            scratch.append(pltpu.VMEM((_CH, widths[t]), jnp.float32))
    scratch.append(pltpu.SemaphoreType.DMA)
    scratch += [pltpu.SemaphoreType.DMA, pltpu.SemaphoreType.DMA]

    @functools.partial(
        pl.kernel,
        out_type=[jax.ShapeDtypeStruct((_E, w), jnp.float32) for w in widths],
        mesh=mesh,
        scratch_types=scratch,
        compiler_params=pltpu.CompilerParams(use_tc_tiling_on_sc=False),
    )
    def k(*refs):
        tabs = refs[:nt]
        idx_hbm = refs[nt]
        outs = refs[nt + 1:nt + 1 + nt]
        idx_v = refs[nt + 1 + nt]
        bufs = [[refs[nt + 2 + nt + 2 * t + b] for b in range(2)]
                for t in range(nt)]
        gsem = refs[-3]
        wsem = [refs[-2], refs[-1]]

        c = lax.axis_index("c")
        s = lax.axis_index("s")
        w = s * 2 + c
        start = per_w * w + jnp.minimum(w, rem)
        count = per_w + (w < rem).astype(jnp.int32)

        pltpu.sync_copy(idx_hbm.at[d0, pl.ds(start * _CH, per_w * _CH)],
                        idx_v.at[pl.ds(8, per_w * _CH)])

        @pl.when(w < rem)
        def _():
            pltpu.sync_copy(
                idx_hbm.at[d0, pl.ds((start + per_w) * _CH, _CH)],
                idx_v.at[pl.ds(8 + per_w * _CH, _CH)])

        if swap:
            _swap_pairs(idx_v, (per_w + 1) * _CH)

        def chunk(k_, b):
            @pl.when(k_ < count)
            def _():
                @pl.when(k_ >= 2)
                def _():
                    for t in range(nt):
                        pltpu.make_async_copy(
                            bufs[t][b], outs[t].at[pl.ds(0, _CH)],
                            wsem[b]).wait()

                row = idx_v.at[pl.ds(8 + k_ * _CH, _CH)]
                descs = [
                    pltpu.async_copy(tabs[t].at[row], bufs[t][b], gsem)
                    for t in range(nt)
                ]
                for dsc in descs:
                    dsc.wait()
                base = (start + k_) * _CH
                for t in range(nt):
                    pltpu.async_copy(bufs[t][b],
                                     outs[t].at[pl.ds(base, _CH)], wsem[b])

        def outer(k2, carry):
            chunk(k2 * 2, 0)
            chunk(k2 * 2 + 1, 1)
            return carry

        lax.fori_loop(0, (per_w + 2) // 2, outer, 0)

        for b in range(2):
            for t in range(nt):
                pltpu.make_async_copy(bufs[t][b], outs[t].at[pl.ds(0, _CH)],
                                      wsem[b]).wait()

    return k(*tables, ei)


def _sc_scatter_add(msg3, ei, swap, zrows):
    per_t = _NCHUNK_S // 16
    rem = _NCHUNK_S - per_t * 16
    mesh = plsc.VectorSubcoreMesh(core_axis_name="c", subcore_axis_name="s")

    @functools.partial(
        pl.kernel,
        out_type=[jax.ShapeDtypeStruct((_N, _PW), jnp.float32)
                  for _ in range(_NP)],
        mesh=mesh,
        scratch_types=[
            pltpu.VMEM(((per_t + 1) * _SCH + 16,), jnp.int32),
            pltpu.VMEM((_SCH, _PW), jnp.float32),
            pltpu.VMEM((_SCH, _PW), jnp.float32),
            pltpu.VMEM_SHARED((_N, _PW), jnp.float32),
            pltpu.SemaphoreType.DMA,
            pltpu.SemaphoreType.DMA,
            pltpu.SemaphoreType.DMA,
            pltpu.SemaphoreType.DMA,
        ],
        compiler_params=pltpu.CompilerParams(use_tc_tiling_on_sc=False),
    )
    def k(msg_hbm, idx_hbm, z_hbm, o0, o1, o2, idx_v, mv0, mv1, acc,
          l0, l1, a0, a1):
        c = lax.axis_index("c")
        s = lax.axis_index("s")
        outs = [o0, o1, o2]
        mv = [mv0, mv1]
        lsem = [l0, l1]
        asem = [a0, a1]
        start = per_t * s + jnp.minimum(s, rem)
        count = per_t + (s < rem).astype(jnp.int32)

        pltpu.sync_copy(idx_hbm.at[1, pl.ds(start * _SCH, per_t * _SCH)],
                        idx_v.at[pl.ds(8, per_t * _SCH)])

        @pl.when(s < rem)
        def _():
            pltpu.sync_copy(
                idx_hbm.at[1, pl.ds((start + per_t) * _SCH, _SCH)],
                idx_v.at[pl.ds(8 + per_t * _SCH, _SCH)])

        if swap:
            _swap_pairs(idx_v, (per_t + 1) * _SCH)

        for p in range(_NP):
            @pl.when(c == _OWNER[p])
            def _():
                pltpu.sync_copy(z_hbm, acc.at[pl.ds(s * _ZSTR, _ZCH)])
                plsc.subcore_barrier()

                def chunk(k_, b):
                    @pl.when(k_ < count)
                    def _():
                        @pl.when(k_ >= 2)
                        def _():
                            pltpu.make_async_copy(
                                mv[b], acc.at[pl.ds(0, _SCH)],
                                asem[b]).wait()
                        pltpu.async_copy(
                            msg_hbm.at[p, pl.ds((start + k_) * _SCH, _SCH)],
                            mv[b], lsem[b])
                        pltpu.make_async_copy(
                            msg_hbm.at[p, pl.ds(0, _SCH)], mv[b],
                            lsem[b]).wait()
                        pltpu.async_copy(
                            mv[b],
                            acc.at[idx_v.at[pl.ds(8 + k_ * _SCH, _SCH)]],
                            asem[b], add=True)

                def outer(k2, carry):
                    chunk(k2 * 2, 0)
                    chunk(k2 * 2 + 1, 1)
                    return carry

                lax.fori_loop(0, (per_t + 2) // 2, outer, 0)

                for b in range(2):
                    pltpu.make_async_copy(mv[b], acc.at[pl.ds(0, _SCH)],
                                          asem[b]).wait()

                plsc.subcore_barrier()

                out = outs[p]

                @pl.when(s < 15)
                def _():
                    pltpu.sync_copy(acc.at[pl.ds(s * _FCH, _FCH)],
                                    out.at[pl.ds(s * _FCH, _FCH)])

                @pl.when(s == 15)
                def _():
                    pltpu.sync_copy(
                        acc.at[pl.ds(15 * _FCH, _N - 15 * _FCH)],
                        out.at[pl.ds(15 * _FCH, _N - 15 * _FCH)])

                plsc.subcore_barrier()

    return k(msg3, ei, zrows)


_BE = 640
_GRID_E = _E // _BE


def _silu(x):
    return x * jax.nn.sigmoid(x)


def _split_store(m3_ref, m):
    for p in range(_NP):
        m3_ref[p] = m[:, p * _PW:(p + 1) * _PW]


def _tc_edge_init(xg, ea, WeX, WeE, be2, W0, b0):
    def body(xg_ref, ea_ref, wx_ref, we_ref, be_ref, w0_ref, b0_ref,
             e_ref, m_ref):
        h = jnp.dot(xg_ref[...], wx_ref[...], preferred_element_type=jnp.float32)
        h = h + jnp.dot(ea_ref[...], we_ref[...], preferred_element_type=jnp.float32)
        h = h + be_ref[...]
        e = _silu(h)
        e_ref[...] = e
        m = jnp.dot(e, w0_ref[...], preferred_element_type=jnp.float32) + b0_ref[...]
        _split_store(m_ref, jnp.maximum(m, 0.0))

    full = lambda shape: pl.BlockSpec(shape, lambda i: (0, 0))
    return pl.pallas_call(
        body,
        grid=(_GRID_E,),
        in_specs=[
            pl.BlockSpec((_BE, 128), lambda i: (i, 0)),
            pl.BlockSpec((_BE, 16), lambda i: (i, 0)),
            full((128, _HP)),
            full((16, _HP)),
            full((1, _HP)),
            full((_HP, _HP)),
            full((1, _HP)),
        ],
        out_specs=[
            pl.BlockSpec((_BE, _HP), lambda i: (i, 0)),
            pl.BlockSpec((_NP, _BE, _PW), lambda i: (0, i, 0)),
        ],
        out_shape=[
            jax.ShapeDtypeStruct((_E, _HP), jnp.float32),
            jax.ShapeDtypeStruct((_NP, _E, _PW), jnp.float32),
        ],
        compiler_params=pltpu.CompilerParams(
            dimension_semantics=("parallel",)),
    )(xg, ea, WeX, WeE, be2, W0, b0)


def _tc_layer(agA, agB, agC, ep, mW, mb2, nW, nb2, last):
    def body(aga_ref, agb_ref, agc_ref, ep_ref, mw_ref, mb_ref, nw_ref,
             nb_ref, *outs):
        ag = jnp.concatenate(
            [aga_ref[...], agb_ref[...], agc_ref[...]], axis=1)
        d = ag - ep_ref[...]
        h = jnp.dot(d, mw_ref[...], preferred_element_type=jnp.float32) + mb_ref[...]
        h = jnp.maximum(h, 0.0)
        if last:
            e = 2.0 * h
            m_ref, = outs
        else:
            e = _silu(h) + h
            e_ref, m_ref = outs
            e_ref[...] = e
        m = jnp.dot(e, nw_ref[...], preferred_element_type=jnp.float32) + nb_ref[...]
        _split_store(m_ref, jnp.maximum(m, 0.0))

    full = lambda shape: pl.BlockSpec(shape, lambda i: (0, 0))
    out_specs = [pl.BlockSpec((_NP, _BE, _PW), lambda i: (0, i, 0))]
    out_shape = [jax.ShapeDtypeStruct((_NP, _E, _PW), jnp.float32)]
    if not last:
        out_specs = [pl.BlockSpec((_BE, _HP), lambda i: (i, 0))] + out_specs
        out_shape = [jax.ShapeDtypeStruct((_E, _HP), jnp.float32)] + out_shape
    return pl.pallas_call(
        body,
        grid=(_GRID_E,),
        in_specs=[
            pl.BlockSpec((_BE, _PW), lambda i: (i, 0)),
            pl.BlockSpec((_BE, _PW), lambda i: (i, 0)),
            pl.BlockSpec((_BE, _PW), lambda i: (i, 0)),
            pl.BlockSpec((_BE, _HP), lambda i: (i, 0)),
            full((_HP, _HP)),
            full((1, _HP)),
            full((_HP, _HP)),
            full((1, _HP)),
        ],
        out_specs=out_specs,
        out_shape=out_shape,
        compiler_params=pltpu.CompilerParams(
            dimension_semantics=("parallel",)),
    )(agA, agB, agC, ep, mW, mb2, nW, nb2)


_BN = 400
_GRID_N = _N // _BN


def _tc_pool_ffn(nhA, nhB, nhC, batch3, W1p, b12, W2, b22, W3, b32):
    def body(nha_ref, nhb_ref, nhc_ref, b_ref, w1_ref, b1_ref, w2_ref,
             b2_ref, w3_ref, b3_ref, out_ref, acc):
        i = pl.program_id(0)
        nh = jnp.concatenate(
            [nha_ref[...], nhb_ref[...], nhc_ref[...]], axis=1)
        seg = jnp.broadcast_to(b_ref[0], (_G, _BN))
        gids = lax.broadcasted_iota(jnp.int32, (_G, _BN), 0)
        onehot = (seg == gids).astype(jnp.float32)
        part = jnp.dot(onehot, nh, preferred_element_type=jnp.float32)

        @pl.when(i == 0)
        def _():
            acc[...] = jnp.zeros_like(acc)

        acc[...] += part

        @pl.when(i == _GRID_N - 1)
        def _():
            h = jnp.dot(acc[...], w1_ref[...], preferred_element_type=jnp.float32)
            h = _silu(h + b1_ref[...])
            h = jnp.dot(h, w2_ref[...], preferred_element_type=jnp.float32)
            h = _silu(h + b2_ref[...])
            h = jnp.dot(h, w3_ref[...], preferred_element_type=jnp.float32)
            out_ref[...] = h + b3_ref[...]

    full = lambda shape: pl.BlockSpec(shape, lambda i: tuple(0 for _ in shape))
    return pl.pallas_call(
        body,
        grid=(_GRID_N,),
        in_specs=[
            pl.BlockSpec((_BN, _PW), lambda i: (i, 0)),
            pl.BlockSpec((_BN, _PW), lambda i: (i, 0)),
            pl.BlockSpec((_BN, _PW), lambda i: (i, 0)),
            pl.BlockSpec((1, 1, _BN), lambda i: (i, 0, 0)),
            full((_HP, 300)),
            full((1, 300)),
            full((300, 300)),
            full((1, 300)),
            full((300, 1)),
            full((1, 1)),
        ],
        out_specs=pl.BlockSpec((_G, 1), lambda i: (0, 0)),
        out_shape=jax.ShapeDtypeStruct((_G, 1), jnp.float32),
        scratch_shapes=[pltpu.VMEM((_G, _HP), jnp.float32)],
        compiler_params=pltpu.CompilerParams(
            dimension_semantics=("arbitrary",)),
    )(nhA, nhB, nhC, batch3, W1p, b12, W2, b22, W3, b32)


def kernel(x, edge_index, edge_attr, batch, We, be, linW, linb, mlpW, mlpb,
           n2W, n2b, W1, b1, W2, b2, W3, b3):
    f32 = jnp.float32
    ei = edge_index.astype(jnp.int32)

    pad_h = lambda w: jnp.pad(w, ((0, _HP - w.shape[0]), (0, _HP - w.shape[1])))
    pad_b = lambda v: jnp.pad(v, (0, _HP - v.shape[0])).reshape(1, _HP)

    WeX = jnp.pad(We[:128], ((0, 0), (0, _HP - 300)))
    WeE = jnp.pad(We[128:], ((0, 0), (0, _HP - 300)))
    be2 = pad_b(be)
    linWp = [pad_h(linW[l]) for l in range(3)]
    linbp = [pad_b(linb[l]) for l in range(3)]
    mlpWp = [pad_h(mlpW[l]) for l in range(3)]
    mlpbp = [pad_b(mlpb[l]) for l in range(3)]
    n2Wp = pad_h(n2W)
    n2bp = pad_b(n2b)
    W1p = jnp.pad(W1, ((0, _HP - 300), (0, 0)))
    b12 = b1.reshape(1, 300)
    b22 = b2.reshape(1, 300)
    b32 = b3.reshape(1, 1)
    batch3 = batch.astype(jnp.int32).reshape(_GRID_N, 1, _BN)
    zrows = jnp.zeros((_ZCH, _PW), f32)

    xg, = _sc_gather([x], ei, 0, False, [128])
    e0, msg0 = _tc_edge_init(xg, edge_attr, WeX, WeE, be2, linWp[0], linbp[0])

    a0 = _sc_scatter_add(msg0, ei, False, zrows)
    ag0 = _sc_gather(a0, ei, 0, True, [_PW] * _NP)
    e1s, msg1s = _tc_layer(*ag0, e0, mlpWp[0], mlpbp[0],
                           linWp[1], linbp[1], False)

    a1 = _sc_scatter_add(msg1s, ei, True, zrows)
    ag1 = _sc_gather(a1, ei, 0, False, [_PW] * _NP)
    e2, msg2 = _tc_layer(*ag1, e1s, mlpWp[1], mlpbp[1],
                         linWp[2], linbp[2], False)

    a2 = _sc_scatter_add(msg2, ei, False, zrows)
    ag2 = _sc_gather(a2, ei, 0, True, [_PW] * _NP)
    msgf, = _tc_layer(*ag2, e2, mlpWp[2], mlpbp[2], n2Wp, n2bp, True)

    nh = _sc_scatter_add(msgf, ei, True, zrows)
    return _tc_pool_ffn(*nh, batch3, W1p, b12, W2, b22, W3, b32)


# speedup vs baseline: 1.9007x; 1.0372x over previous
import functools

import jax
import jax.numpy as jnp
from jax import lax
from jax.experimental import pallas as pl
from jax.experimental.pallas import tpu as pltpu
from jax.experimental.pallas import tpu_sc as plsc

_N = 10000
_E = 160000
_PW = 128
_NP = 3
_HP = _PW * _NP
_G = 64
_CH = 128
_NCHUNK = _E // _CH
_SCH = 64
_NCHUNK_S = _E // _SCH
_NW = 32
_ZSTR = 624
_ZCH = 640
_FCH = 632
_OWNER = (0, 1, 0)


def _swap_pairs(idx_v, n):
    par = (jax.lax.iota(jnp.int32, 16) & 1) == 0

    def s16(i, carry):
        o = 8 + i * 16
        a = idx_v[pl.ds(o + 1, 16)]
        b = idx_v[pl.ds(o - 1, 16)]
        idx_v[pl.ds(o, 16)] = jnp.where(par, a, b)
        return carry

    lax.fori_loop(0, n // 16, s16, 0)


def _sc_gather(tables, ei, d0, swap, widths):
    nt = len(tables)
    per_w = _NCHUNK // _NW
    rem = _NCHUNK - per_w * _NW
    mesh = plsc.VectorSubcoreMesh(core_axis_name="c", subcore_axis_name="s")

    scratch = [pltpu.VMEM(((per_w + 1) * _CH + 16,), jnp.int32)]
    for t in range(nt):
        for b in range(2):
            scratch.append(pltpu.VMEM((_CH, widths[t]), jnp.float32))
    scratch.append(pltpu.SemaphoreType.DMA)
    scratch += [pltpu.SemaphoreType.DMA, pltpu.SemaphoreType.DMA]

    @functools.partial(
        pl.kernel,
        out_type=[jax.ShapeDtypeStruct((_E, w), jnp.float32) for w in widths],
        mesh=mesh,
        scratch_types=scratch,
        compiler_params=pltpu.CompilerParams(use_tc_tiling_on_sc=False),
    )
    def k(*refs):
        tabs = refs[:nt]
        idx_hbm = refs[nt]
        outs = refs[nt + 1:nt + 1 + nt]
        idx_v = refs[nt + 1 + nt]
        bufs = [[refs[nt + 2 + nt + 2 * t + b] for b in range(2)]
                for t in range(nt)]
        gsem = refs[-3]
        wsem = [refs[-2], refs[-1]]

        c = lax.axis_index("c")
        s = lax.axis_index("s")
        w = s * 2 + c
        start = per_w * w + jnp.minimum(w, rem)
        count = per_w + (w < rem).astype(jnp.int32)

        pltpu.sync_copy(idx_hbm.at[d0, pl.ds(start * _CH, per_w * _CH)],
                        idx_v.at[pl.ds(8, per_w * _CH)])

        @pl.when(w < rem)
        def _():
            pltpu.sync_copy(
                idx_hbm.at[d0, pl.ds((start + per_w) * _CH, _CH)],
                idx_v.at[pl.ds(8 + per_w * _CH, _CH)])

        if swap:
            _swap_pairs(idx_v, (per_w + 1) * _CH)

        def chunk(k_, b):
            @pl.when(k_ < count)
            def _():
                @pl.when(k_ >= 2)
                def _():
                    for t in range(nt):
                        pltpu.make_async_copy(
                            bufs[t][b], outs[t].at[pl.ds(0, _CH)],
                            wsem[b]).wait()

                row = idx_v.at[pl.ds(8 + k_ * _CH, _CH)]
                descs = [
                    pltpu.async_copy(tabs[t].at[row], bufs[t][b], gsem)
                    for t in range(nt)
                ]
                for dsc in descs:
                    dsc.wait()
                base = (start + k_) * _CH
                for t in range(nt):
                    pltpu.async_copy(bufs[t][b],
                                     outs[t].at[pl.ds(base, _CH)], wsem[b])

        def outer(k2, carry):
            chunk(k2 * 2, 0)
            chunk(k2 * 2 + 1, 1)
            return carry

        lax.fori_loop(0, (per_w + 2) // 2, outer, 0)

        for b in range(2):
            for t in range(nt):
                pltpu.make_async_copy(bufs[t][b], outs[t].at[pl.ds(0, _CH)],
                                      wsem[b]).wait()

    return k(*tables, ei)


def _sc_scatter_add(msg3, ei, swap, zrows):
    per_t = _NCHUNK_S // 16
    rem = _NCHUNK_S - per_t * 16
    mesh = plsc.VectorSubcoreMesh(core_axis_name="c", subcore_axis_name="s")

    @functools.partial(
        pl.kernel,
        out_type=[jax.ShapeDtypeStruct((_N, _PW), jnp.float32)
                  for _ in range(_NP)],
        mesh=mesh,
        scratch_types=[
            pltpu.VMEM(((per_t + 1) * _SCH + 16,), jnp.int32),
            pltpu.VMEM((_SCH, _PW), jnp.float32),
            pltpu.VMEM((_SCH, _PW), jnp.float32),
            pltpu.VMEM_SHARED((_N, _PW), jnp.float32),
            pltpu.SemaphoreType.DMA,
            pltpu.SemaphoreType.DMA,
            pltpu.SemaphoreType.DMA,
            pltpu.SemaphoreType.DMA,
        ],
        compiler_params=pltpu.CompilerParams(use_tc_tiling_on_sc=False),
    )
    def k(msg_hbm, idx_hbm, z_hbm, o0, o1, o2, idx_v, mv0, mv1, acc,
          l0, l1, a0, a1):
        c = lax.axis_index("c")
        s = lax.axis_index("s")
        outs = [o0, o1, o2]
        mv = [mv0, mv1]
        lsem = [l0, l1]
        asem = [a0, a1]
        start = per_t * s + jnp.minimum(s, rem)
        count = per_t + (s < rem).astype(jnp.int32)

        pltpu.sync_copy(idx_hbm.at[1, pl.ds(start * _SCH, per_t * _SCH)],
                        idx_v.at[pl.ds(8, per_t * _SCH)])

        @pl.when(s < rem)
        def _():
            pltpu.sync_copy(
                idx_hbm.at[1, pl.ds((start + per_t) * _SCH, _SCH)],
                idx_v.at[pl.ds(8 + per_t * _SCH, _SCH)])

        if swap:
            _swap_pairs(idx_v, (per_t + 1) * _SCH)

        for p in range(_NP):
            @pl.when(c == _OWNER[p])
            def _():
                pltpu.sync_copy(z_hbm, acc.at[pl.ds(s * _ZSTR, _ZCH)])
                plsc.subcore_barrier()

                def chunk(k_, b):
                    @pl.when(k_ < count)
                    def _():
                        @pl.when(k_ >= 2)
                        def _():
                            pltpu.make_async_copy(
                                mv[b], acc.at[pl.ds(0, _SCH)],
                                asem[b]).wait()
                        pltpu.async_copy(
                            msg_hbm.at[p, pl.ds((start + k_) * _SCH, _SCH)],
                            mv[b], lsem[b])
                        pltpu.make_async_copy(
                            msg_hbm.at[p, pl.ds(0, _SCH)], mv[b],
                            lsem[b]).wait()
                        pltpu.async_copy(
                            mv[b],
                            acc.at[idx_v.at[pl.ds(8 + k_ * _SCH, _SCH)]],
                            asem[b], add=True)

                def outer(k2, carry):
                    chunk(k2 * 2, 0)
                    chunk(k2 * 2 + 1, 1)
                    return carry

                lax.fori_loop(0, (per_t + 2) // 2, outer, 0)

                for b in range(2):
                    pltpu.make_async_copy(mv[b], acc.at[pl.ds(0, _SCH)],
                                          asem[b]).wait()

                plsc.subcore_barrier()

                out = outs[p]

                @pl.when(s < 15)
                def _():
                    pltpu.sync_copy(acc.at[pl.ds(s * _FCH, _FCH)],
                                    out.at[pl.ds(s * _FCH, _FCH)])

                @pl.when(s == 15)
                def _():
                    pltpu.sync_copy(
                        acc.at[pl.ds(15 * _FCH, _N - 15 * _FCH)],
                        out.at[pl.ds(15 * _FCH, _N - 15 * _FCH)])

                plsc.subcore_barrier()

    return k(msg3, ei, zrows)


_BE = 640
_GRID_E = _E // _BE


def _silu(x):
    return x * jax.nn.sigmoid(x)


def _split_store(m3_ref, m):
    for p in range(_NP):
        m3_ref[p] = m[:, p * _PW:(p + 1) * _PW]


def _tc_edge_init(xg, ea, WeX, WeE, be2, W0, b0):
    def body(xg_ref, ea_ref, wx_ref, we_ref, be_ref, w0_ref, b0_ref,
             e_ref, m_ref):
        h = jnp.dot(xg_ref[...], wx_ref[...], preferred_element_type=jnp.float32)
        h = h + jnp.dot(ea_ref[...], we_ref[...], preferred_element_type=jnp.float32)
        h = h + be_ref[...]
        e = _silu(h)
        e_ref[...] = e.astype(jnp.bfloat16)
        m = jnp.dot(e, w0_ref[...], preferred_element_type=jnp.float32) + b0_ref[...]
        _split_store(m_ref, jnp.maximum(m, 0.0))

    full = lambda shape: pl.BlockSpec(shape, lambda i: (0, 0))
    return pl.pallas_call(
        body,
        grid=(_GRID_E,),
        in_specs=[
            pl.BlockSpec((_BE, 128), lambda i: (i, 0)),
            pl.BlockSpec((_BE, 16), lambda i: (i, 0)),
            full((128, _HP)),
            full((16, _HP)),
            full((1, _HP)),
            full((_HP, _HP)),
            full((1, _HP)),
        ],
        out_specs=[
            pl.BlockSpec((_BE, _HP), lambda i: (i, 0)),
            pl.BlockSpec((_NP, _BE, _PW), lambda i: (0, i, 0)),
        ],
        out_shape=[
            jax.ShapeDtypeStruct((_E, _HP), jnp.bfloat16),
            jax.ShapeDtypeStruct((_NP, _E, _PW), jnp.float32),
        ],
        compiler_params=pltpu.CompilerParams(
            dimension_semantics=("parallel",)),
    )(xg, ea, WeX, WeE, be2, W0, b0)


def _tc_layer(agA, agB, agC, ep, mW, mb2, nW, nb2, last):
    def body(aga_ref, agb_ref, agc_ref, ep_ref, mw_ref, mb_ref, nw_ref,
             nb_ref, *outs):
        ag = jnp.concatenate(
            [aga_ref[...], agb_ref[...], agc_ref[...]], axis=1)
        d = ag - ep_ref[...].astype(jnp.float32)
        h = jnp.dot(d, mw_ref[...], preferred_element_type=jnp.float32) + mb_ref[...]
        h = jnp.maximum(h, 0.0)
        if last:
            e = 2.0 * h
            m_ref, = outs
        else:
            e = _silu(h) + h
            e_ref, m_ref = outs
            e_ref[...] = e.astype(jnp.bfloat16)
        m = jnp.dot(e, nw_ref[...], preferred_element_type=jnp.float32) + nb_ref[...]
        _split_store(m_ref, jnp.maximum(m, 0.0))

    full = lambda shape: pl.BlockSpec(shape, lambda i: (0, 0))
    out_specs = [pl.BlockSpec((_NP, _BE, _PW), lambda i: (0, i, 0))]
    out_shape = [jax.ShapeDtypeStruct((_NP, _E, _PW), jnp.float32)]
    if not last:
        out_specs = [pl.BlockSpec((_BE, _HP), lambda i: (i, 0))] + out_specs
        out_shape = [jax.ShapeDtypeStruct((_E, _HP), jnp.bfloat16)] + out_shape
    return pl.pallas_call(
        body,
        grid=(_GRID_E,),
        in_specs=[
            pl.BlockSpec((_BE, _PW), lambda i: (i, 0)),
            pl.BlockSpec((_BE, _PW), lambda i: (i, 0)),
            pl.BlockSpec((_BE, _PW), lambda i: (i, 0)),
            pl.BlockSpec((_BE, _HP), lambda i: (i, 0)),
            full((_HP, _HP)),
            full((1, _HP)),
            full((_HP, _HP)),
            full((1, _HP)),
        ],
        out_specs=out_specs,
        out_shape=out_shape,
        compiler_params=pltpu.CompilerParams(
            dimension_semantics=("parallel",)),
    )(agA, agB, agC, ep, mW, mb2, nW, nb2)


_BN = 400
_GRID_N = _N // _BN


def _tc_pool_ffn(nhA, nhB, nhC, batch3, W1p, b12, W2, b22, W3, b32):
    def body(nha_ref, nhb_ref, nhc_ref, b_ref, w1_ref, b1_ref, w2_ref,
             b2_ref, w3_ref, b3_ref, out_ref, acc):
        i = pl.program_id(0)
        nh = jnp.concatenate(
            [nha_ref[...], nhb_ref[...], nhc_ref[...]], axis=1)
        seg = jnp.broadcast_to(b_ref[0], (_G, _BN))
        gids = lax.broadcasted_iota(jnp.int32, (_G, _BN), 0)
        onehot = (seg == gids).astype(jnp.float32)
        part = jnp.dot(onehot, nh, preferred_element_type=jnp.float32)

        @pl.when(i == 0)
        def _():
            acc[...] = jnp.zeros_like(acc)

        acc[...] += part

        @pl.when(i == _GRID_N - 1)
        def _():
            h = jnp.dot(acc[...], w1_ref[...], preferred_element_type=jnp.float32)
            h = _silu(h + b1_ref[...])
            h = jnp.dot(h, w2_ref[...], preferred_element_type=jnp.float32)
            h = _silu(h + b2_ref[...])
            h = jnp.dot(h, w3_ref[...], preferred_element_type=jnp.float32)
            out_ref[...] = h + b3_ref[...]

    full = lambda shape: pl.BlockSpec(shape, lambda i: tuple(0 for _ in shape))
    return pl.pallas_call(
        body,
        grid=(_GRID_N,),
        in_specs=[
            pl.BlockSpec((_BN, _PW), lambda i: (i, 0)),
            pl.BlockSpec((_BN, _PW), lambda i: (i, 0)),
            pl.BlockSpec((_BN, _PW), lambda i: (i, 0)),
            pl.BlockSpec((1, 1, _BN), lambda i: (i, 0, 0)),
            full((_HP, 300)),
            full((1, 300)),
            full((300, 300)),
            full((1, 300)),
            full((300, 1)),
            full((1, 1)),
        ],
        out_specs=pl.BlockSpec((_G, 1), lambda i: (0, 0)),
        out_shape=jax.ShapeDtypeStruct((_G, 1), jnp.float32),
        scratch_shapes=[pltpu.VMEM((_G, _HP), jnp.float32)],
        compiler_params=pltpu.CompilerParams(
            dimension_semantics=("arbitrary",)),
    )(nhA, nhB, nhC, batch3, W1p, b12, W2, b22, W3, b32)


def kernel(x, edge_index, edge_attr, batch, We, be, linW, linb, mlpW, mlpb,
           n2W, n2b, W1, b1, W2, b2, W3, b3):
    f32 = jnp.float32
    ei = edge_index.astype(jnp.int32)

    pad_h = lambda w: jnp.pad(w, ((0, _HP - w.shape[0]), (0, _HP - w.shape[1])))
    pad_b = lambda v: jnp.pad(v, (0, _HP - v.shape[0])).reshape(1, _HP)

    WeX = jnp.pad(We[:128], ((0, 0), (0, _HP - 300)))
    WeE = jnp.pad(We[128:], ((0, 0), (0, _HP - 300)))
    be2 = pad_b(be)
    linWp = [pad_h(linW[l]) for l in range(3)]
    linbp = [pad_b(linb[l]) for l in range(3)]
    mlpWp = [pad_h(mlpW[l]) for l in range(3)]
    mlpbp = [pad_b(mlpb[l]) for l in range(3)]
    n2Wp = pad_h(n2W)
    n2bp = pad_b(n2b)
    W1p = jnp.pad(W1, ((0, _HP - 300), (0, 0)))
    b12 = b1.reshape(1, 300)
    b22 = b2.reshape(1, 300)
    b32 = b3.reshape(1, 1)
    batch3 = batch.astype(jnp.int32).reshape(_GRID_N, 1, _BN)
    zrows = jnp.zeros((_ZCH, _PW), f32)

    xg, = _sc_gather([x], ei, 0, False, [128])
    e0, msg0 = _tc_edge_init(xg, edge_attr, WeX, WeE, be2, linWp[0], linbp[0])

    a0 = _sc_scatter_add(msg0, ei, False, zrows)
    ag0 = _sc_gather(a0, ei, 0, True, [_PW] * _NP)
    e1s, msg1s = _tc_layer(*ag0, e0, mlpWp[0], mlpbp[0],
                           linWp[1], linbp[1], False)

    a1 = _sc_scatter_add(msg1s, ei, True, zrows)
    ag1 = _sc_gather(a1, ei, 0, False, [_PW] * _NP)
    e2, msg2 = _tc_layer(*ag1, e1s, mlpWp[1], mlpbp[1],
                         linWp[2], linbp[2], False)

    a2 = _sc_scatter_add(msg2, ei, False, zrows)
    ag2 = _sc_gather(a2, ei, 0, True, [_PW] * _NP)
    msgf, = _tc_layer(*ag2, e2, mlpWp[2], mlpbp[2], n2Wp, n2bp, True)

    nh = _sc_scatter_add(msgf, ei, True, zrows)
    return _tc_pool_ffn(*nh, batch3, W1p, b12, W2, b22, W3, b32)
